# Initial kernel scaffold; baseline (speedup 1.0000x reference)
import jax
import jax.numpy as jnp
from jax.experimental import pallas as pl


def kernel(x, edge_index, W0, b0, W1, b1, W2, b2, gamma0, beta0, gamma1, beta1, a_mu, a_log_sigma):
    raise NotImplementedError("write your pallas kernel here")


# speedup vs baseline: 3.2571x; 3.2571x over previous
import functools

import jax
import jax.numpy as jnp
from jax import lax
from jax.experimental import pallas as pl
from jax.experimental.pallas import tpu as pltpu
from jax.experimental.pallas import tpu_sc as plsc

NC, NS, L = 2, 16, 16
NW = NC * NS
CB = 80


def _sc_mesh():
    return plsc.VectorSubcoreMesh(core_axis_name="c", subcore_axis_name="s")


def _zero_vmem(ref, rows, width):
    zero = jnp.zeros((L,), jnp.float32)

    def body(i, carry):
        for j in range(width // L):
            ref[i, pl.ds(j * L, L)] = zero
        return carry

    lax.fori_loop(0, rows, body, 0)


@functools.partial(jax.jit, static_argnames=("n", "e", "dw"))
def _sc_spmm(g, src, dst, a, *, n, e, dw):
    t = e // NW
    nch = t // CB
    npad = ((n + NS * 128 - 1) // (NS * 128)) * (NS * 128)
    rpt = npad // NS
    wb = 128
    nwb = rpt // wb

    @functools.partial(
        pl.kernel,
        out_type=jax.ShapeDtypeStruct((NC, npad, dw), jnp.float32),
        mesh=_sc_mesh(),
        compiler_params=pltpu.CompilerParams(needs_layout_passes=False),
        scratch_types=[
            pltpu.VMEM((CB,), jnp.int32),
            pltpu.VMEM((CB,), jnp.int32),
            pltpu.VMEM((CB,), jnp.float32),
            pltpu.VMEM((CB, dw), jnp.float32),
            pltpu.VMEM((wb, dw), jnp.float32),
            pltpu.VMEM_SHARED((npad, dw), jnp.float32),
            pltpu.SemaphoreType.DMA,
        ],
    )
    def spmm_kernel(g_hbm, src_hbm, dst_hbm, a_hbm, out_hbm,
                    src_v, dst_v, a_v, rows_v, wb_v, acc, sem):
        c = lax.axis_index("c")
        s = lax.axis_index("s")
        wid = c * NS + s
        base = wid * t
        row0 = s * rpt

        _zero_vmem(wb_v, wb, dw)
        for i in range(nwb):
            pltpu.sync_copy(wb_v, acc.at[pl.ds(row0 + i * wb, wb)])
        plsc.subcore_barrier()

        def chunk(k, carry):
            off = pl.multiple_of(base + k * CB, 8)
            pltpu.sync_copy(src_hbm.at[pl.ds(off, CB)], src_v)
            pltpu.sync_copy(dst_hbm.at[pl.ds(off, CB)], dst_v)
            pltpu.sync_copy(a_hbm.at[pl.ds(off, CB)], a_v)
            pltpu.async_copy(g_hbm.at[src_v], rows_v, sem).wait()

            def scale(i, inner):
                ab = plsc.load_gather(
                    a_v, [jnp.zeros((L,), jnp.int32) + i])
                for j in range(dw // L):
                    rows_v[i, pl.ds(j * L, L)] = (
                        rows_v[i, pl.ds(j * L, L)] * ab)
                return inner

            lax.fori_loop(0, CB, scale, 0)
            pltpu.sync_copy(rows_v, acc.at[dst_v], add=True)
            return carry

        lax.fori_loop(0, nch, chunk, 0)
        plsc.subcore_barrier()

        for i in range(nwb):
            r = row0 + i * wb
            pltpu.sync_copy(acc.at[pl.ds(r, wb)], wb_v)
            pltpu.sync_copy(wb_v, out_hbm.at[c, pl.ds(r, wb)])

    return spmm_kernel(g, src, dst, a)


def _tc_prep(degoP, degiP, x, eps, a_mu, a_log_sigma):
    n, d = x.shape

    def body(dego_ref, degi_ref, x_ref, eps_ref, amu_ref, als_ref,
             invo_ref, invi_ref, g0_ref, a_ref):
        dego = dego_ref[0, :n, 0:1] + dego_ref[1, :n, 0:1]
        degi = degi_ref[0, :n, 0:1] + degi_ref[1, :n, 0:1]
        invo = lax.rsqrt(jnp.where(dego > 0.0, dego, 1.0))
        invi = lax.rsqrt(jnp.where(degi > 0.0, degi, 1.0))
        invo_ref[...] = invo
        invi_ref[...] = invi
        g0_ref[...] = x_ref[...] * invo
        sigma = jnp.exp(als_ref[0, 0])
        a_ref[...] = amu_ref[0, 0] + sigma * eps_ref[...]

    return pl.pallas_call(
        body,
        out_shape=(
            jax.ShapeDtypeStruct((n, 1), jnp.float32),
            jax.ShapeDtypeStruct((n, 1), jnp.float32),
            jax.ShapeDtypeStruct((n, d), jnp.float32),
            jax.ShapeDtypeStruct(eps.shape, jnp.float32),
        ),
    )(degoP, degiP, x, eps, a_mu.reshape(1, 1), a_log_sigma.reshape(1, 1))


def _tc_layer(aggP, invi, invo, W, b, gamma, beta, W2=None):
    n = invi.shape[0]
    hdim = W.shape[1]
    odim = hdim if W2 is None else W2.shape[1]

    def body(*refs):
        if W2 is None:
            aggp_ref, invi_ref, invo_ref, w_ref, b_ref, g_ref, be_ref, o_ref = refs
        else:
            (aggp_ref, invi_ref, invo_ref, w_ref, b_ref, g_ref, be_ref,
             w2_ref, o_ref) = refs
        agg = (aggp_ref[0, :n] + aggp_ref[1, :n]) * invi_ref[...]
        tmat = jnp.dot(agg, w_ref[...], preferred_element_type=jnp.float32)
        tmat = tmat + b_ref[...]
        tmat = jnp.maximum(tmat, 0.0)
        mu = jnp.mean(tmat, axis=0, keepdims=True)
        var = jnp.mean((tmat - mu) * (tmat - mu), axis=0, keepdims=True)
        h = (tmat - mu) * lax.rsqrt(var + 1e-5) * g_ref[...] + be_ref[...]
        h = h * invo_ref[...]
        if W2 is None:
            o_ref[...] = h
        else:
            o_ref[...] = jnp.dot(h, w2_ref[...],
                                 preferred_element_type=jnp.float32)

    args = [aggP, invi, invo, W, b.reshape(1, hdim), gamma.reshape(1, hdim),
            beta.reshape(1, hdim)]
    if W2 is not None:
        args.append(W2)
    return pl.pallas_call(
        body,
        out_shape=jax.ShapeDtypeStruct((n, odim), jnp.float32),
    )(*args)


def _tc_final(aggP, invi, b2):
    n = invi.shape[0]
    cdim = b2.shape[0]

    def body(aggp_ref, invi_ref, b_ref, o_ref):
        z = ((aggp_ref[0, :n, :cdim] + aggp_ref[1, :n, :cdim])
             * invi_ref[...] + b_ref[...])
        m = jnp.max(z, axis=1, keepdims=True)
        ez = jnp.exp(z - m)
        o_ref[...] = ez / jnp.sum(ez, axis=1, keepdims=True)

    return pl.pallas_call(
        body,
        out_shape=jax.ShapeDtypeStruct((n, cdim), jnp.float32),
    )(aggP, invi, b2.reshape(1, cdim))


def kernel(x, edge_index, W0, b0, W1, b1, W2, b2,
           gamma0, beta0, gamma1, beta1, a_mu, a_log_sigma):
    n, d = x.shape
    e = edge_index.shape[1]
    src = edge_index[0].astype(jnp.int32)
    dst = edge_index[1].astype(jnp.int32)

    eps = jnp.stack([
        jax.random.normal(jax.random.key(101), (e,), dtype=jnp.float32),
        jax.random.normal(jax.random.key(102), (e,), dtype=jnp.float32),
        jax.random.normal(jax.random.key(103), (e,), dtype=jnp.float32),
    ])

    ones_g = jnp.ones((n, d), jnp.float32)
    ones_a = jnp.ones((e,), jnp.float32)
    degiP = _sc_spmm(ones_g, src, dst, ones_a, n=n, e=e, dw=d)
    degoP = _sc_spmm(ones_g, dst, src, ones_a, n=n, e=e, dw=d)
    invo, invi, g0, a_all = _tc_prep(degoP, degiP, x, eps, a_mu, a_log_sigma)
    a0, a1, a2 = a_all[0], a_all[1], a_all[2]

    aggP0 = _sc_spmm(g0, src, dst, a0, n=n, e=e, dw=d)
    g1 = _tc_layer(aggP0, invi, invo, W0, b0, gamma0, beta0)
    aggP1 = _sc_spmm(g1, src, dst, a1, n=n, e=e, dw=g1.shape[1])
    w2p = jnp.concatenate([W2, jnp.zeros_like(W2)], axis=1)
    p2 = _tc_layer(aggP1, invi, invo, W1, b1, gamma1, beta1, W2=w2p)
    aggP2 = _sc_spmm(p2, src, dst, a2, n=n, e=e, dw=p2.shape[1])
    return _tc_final(aggP2, invi, b2)


# speedup vs baseline: 3.3797x; 1.0377x over previous
import functools

import jax
import jax.numpy as jnp
from jax import lax
from jax.experimental import pallas as pl
from jax.experimental.pallas import tpu as pltpu
from jax.experimental.pallas import tpu_sc as plsc

NC, NS, L = 2, 16, 16
NW = NC * NS


def _sc_mesh():
    return plsc.VectorSubcoreMesh(core_axis_name="c", subcore_axis_name="s")


def _zero_vmem(ref, rows, width):
    zero = jnp.zeros((L,), jnp.float32)

    def body(i, carry):
        for j in range(width // L):
            ref[i, pl.ds(j * L, L)] = zero
        return carry

    lax.fori_loop(0, rows, body, 0)


CB = 128


@functools.partial(jax.jit, static_argnames=("n", "nch"))
def _sc_spmm(g, src3, dst3, a3, *, n, nch):
    dw = g.shape[1]
    npad = ((n + NS * 128 - 1) // (NS * 128)) * (NS * 128)
    rpt = npad // NS
    wb = 128
    nwb = rpt // wb
    assert nch % 2 == 0

    @functools.partial(
        pl.kernel,
        out_type=jax.ShapeDtypeStruct((NC, npad, dw), jnp.float32),
        mesh=_sc_mesh(),
        compiler_params=pltpu.CompilerParams(needs_layout_passes=False),
        scratch_types=[
            pltpu.VMEM((nch, CB), jnp.int32),
            pltpu.VMEM((CB,), jnp.int32),
            pltpu.VMEM((CB,), jnp.int32),
            pltpu.VMEM((CB,), jnp.float32),
            pltpu.VMEM((CB,), jnp.float32),
            pltpu.VMEM((CB, dw), jnp.float32),
            pltpu.VMEM((CB, dw), jnp.float32),
            pltpu.VMEM_SHARED((npad, dw), jnp.float32),
            pltpu.SemaphoreType.DMA,
            pltpu.SemaphoreType.DMA,
            pltpu.SemaphoreType.DMA,
            pltpu.SemaphoreType.DMA,
        ],
    )
    def spmm_kernel(g_hbm, src_hbm, dst_hbm, a_hbm, out_hbm,
                    src_v, dst_b0, dst_b1, a_b0, a_b1, rows0, rows1,
                    acc, semg0, semg1, semi0, semi1):
        c = lax.axis_index("c")
        s = lax.axis_index("s")
        wid = c * NS + s
        row0 = s * rpt

        _zero_vmem(rows0, wb, dw)
        for i in range(nwb):
            pltpu.sync_copy(rows0, acc.at[pl.ds(row0 + i * wb, wb)])
        pltpu.sync_copy(src_hbm.at[wid], src_v)
        plsc.subcore_barrier()

        def prefetch(k, dst_b, a_b, rows_v, semg, semi):
            pltpu.async_copy(dst_hbm.at[wid, k], dst_b, semi)
            pltpu.async_copy(a_hbm.at[wid, k], a_b, semi)
            pltpu.async_copy(g_hbm.at[src_v.at[k]], rows_v, semg)

        def wait(k, dst_b, a_b, rows_v, semg, semi):
            pltpu.make_async_copy(dst_hbm.at[wid, k], dst_b, semi).wait()
            pltpu.make_async_copy(a_hbm.at[wid, k], a_b, semi).wait()
            pltpu.make_async_copy(g_hbm.at[src_v.at[k]], rows_v, semg).wait()

        def scale(rows_v, a_b):
            def edge(i, carry):
                ab = plsc.load_gather(a_b, [jnp.zeros((L,), jnp.int32) + i])
                for j in range(dw // L):
                    rows_v[i, pl.ds(j * L, L)] = (
                        rows_v[i, pl.ds(j * L, L)] * ab)
                return carry

            lax.fori_loop(0, CB, edge, 0)

        prefetch(0, dst_b0, a_b0, rows0, semg0, semi0)
        prefetch(1, dst_b1, a_b1, rows1, semg1, semi1)

        def pipe(j, carry):
            k0 = 2 * j
            wait(k0, dst_b0, a_b0, rows0, semg0, semi0)
            scale(rows0, a_b0)
            pltpu.sync_copy(rows0, acc.at[dst_b0], add=True)
            prefetch(jnp.minimum(k0 + 2, nch - 2),
                     dst_b0, a_b0, rows0, semg0, semi0)
            k1 = k0 + 1
            wait(k1, dst_b1, a_b1, rows1, semg1, semi1)
            scale(rows1, a_b1)
            pltpu.sync_copy(rows1, acc.at[dst_b1], add=True)
            prefetch(jnp.minimum(k1 + 2, nch - 1),
                     dst_b1, a_b1, rows1, semg1, semi1)
            return carry

        lax.fori_loop(0, nch // 2, pipe, 0)
        wait(nch - 2, dst_b0, a_b0, rows0, semg0, semi0)
        wait(nch - 1, dst_b1, a_b1, rows1, semg1, semi1)
        plsc.subcore_barrier()

        for i in range(nwb):
            r = row0 + i * wb
            pltpu.sync_copy(acc.at[pl.ds(r, wb)], rows0)
            pltpu.sync_copy(rows0, out_hbm.at[c, pl.ds(r, wb)])

    return spmm_kernel(g, src3, dst3, a3)


def _chunk_edges(v, nch):
    pad = NW * nch * CB - v.shape[0]
    return jnp.concatenate(
        [v, jnp.zeros((pad,), v.dtype)]).reshape(NW, nch, CB)


def _tc_prep(degoP, degiP, x, eps, a_mu, a_log_sigma):
    n, d = x.shape

    def body(dego_ref, degi_ref, x_ref, eps_ref, amu_ref, als_ref,
             invo_ref, invi_ref, g0_ref, a_ref):
        dego = dego_ref[0, :n, 0:1] + dego_ref[1, :n, 0:1]
        degi = degi_ref[0, :n, 0:1] + degi_ref[1, :n, 0:1]
        invo = lax.rsqrt(jnp.where(dego > 0.0, dego, 1.0))
        invi = lax.rsqrt(jnp.where(degi > 0.0, degi, 1.0))
        invo_ref[...] = invo
        invi_ref[...] = invi
        g0_ref[...] = x_ref[...] * invo
        sigma = jnp.exp(als_ref[0, 0])
        a_ref[...] = amu_ref[0, 0] + sigma * eps_ref[...]

    return pl.pallas_call(
        body,
        out_shape=(
            jax.ShapeDtypeStruct((n, 1), jnp.float32),
            jax.ShapeDtypeStruct((n, 1), jnp.float32),
            jax.ShapeDtypeStruct((n, d), jnp.float32),
            jax.ShapeDtypeStruct(eps.shape, jnp.float32),
        ),
    )(degoP, degiP, x, eps, a_mu.reshape(1, 1), a_log_sigma.reshape(1, 1))


def _tc_layer(aggP, invi, invo, W, b, gamma, beta, W2=None):
    n = invi.shape[0]
    hdim = W.shape[1]
    odim = hdim if W2 is None else W2.shape[1]

    def body(*refs):
        if W2 is None:
            aggp_ref, invi_ref, invo_ref, w_ref, b_ref, g_ref, be_ref, o_ref = refs
        else:
            (aggp_ref, invi_ref, invo_ref, w_ref, b_ref, g_ref, be_ref,
             w2_ref, o_ref) = refs
        agg = (aggp_ref[0, :n] + aggp_ref[1, :n]) * invi_ref[...]
        tmat = jnp.dot(agg, w_ref[...], preferred_element_type=jnp.float32)
        tmat = tmat + b_ref[...]
        tmat = jnp.maximum(tmat, 0.0)
        mu = jnp.mean(tmat, axis=0, keepdims=True)
        var = jnp.mean((tmat - mu) * (tmat - mu), axis=0, keepdims=True)
        h = (tmat - mu) * lax.rsqrt(var + 1e-5) * g_ref[...] + be_ref[...]
        h = h * invo_ref[...]
        if W2 is None:
            o_ref[...] = h
        else:
            o_ref[...] = jnp.dot(h, w2_ref[...],
                                 preferred_element_type=jnp.float32)

    args = [aggP, invi, invo, W, b.reshape(1, hdim), gamma.reshape(1, hdim),
            beta.reshape(1, hdim)]
    if W2 is not None:
        args.append(W2)
    return pl.pallas_call(
        body,
        out_shape=jax.ShapeDtypeStruct((n, odim), jnp.float32),
    )(*args)


def _tc_final(aggP, invi, b2):
    n = invi.shape[0]
    cdim = b2.shape[0]

    def body(aggp_ref, invi_ref, b_ref, o_ref):
        z = ((aggp_ref[0, :n, :cdim] + aggp_ref[1, :n, :cdim])
             * invi_ref[...] + b_ref[...])
        m = jnp.max(z, axis=1, keepdims=True)
        ez = jnp.exp(z - m)
        o_ref[...] = ez / jnp.sum(ez, axis=1, keepdims=True)

    return pl.pallas_call(
        body,
        out_shape=jax.ShapeDtypeStruct((n, cdim), jnp.float32),
    )(aggP, invi, b2.reshape(1, cdim))


def kernel(x, edge_index, W0, b0, W1, b1, W2, b2,
           gamma0, beta0, gamma1, beta1, a_mu, a_log_sigma):
    n, d = x.shape
    e = edge_index.shape[1]
    src = edge_index[0].astype(jnp.int32)
    dst = edge_index[1].astype(jnp.int32)

    eps = jnp.stack([
        jax.random.normal(jax.random.key(101), (e,), dtype=jnp.float32),
        jax.random.normal(jax.random.key(102), (e,), dtype=jnp.float32),
        jax.random.normal(jax.random.key(103), (e,), dtype=jnp.float32),
    ])

    nch = -(-e // (NW * CB * 2)) * 2
    src3 = _chunk_edges(src, nch)
    dst3 = _chunk_edges(dst, nch)
    ones_g = jnp.ones((n, d), jnp.float32)
    ones_a3 = _chunk_edges(jnp.ones((e,), jnp.float32), nch)
    degiP = _sc_spmm(ones_g, src3, dst3, ones_a3, n=n, nch=nch)
    degoP = _sc_spmm(ones_g, dst3, src3, ones_a3, n=n, nch=nch)
    invo, invi, g0, a_all = _tc_prep(degoP, degiP, x, eps, a_mu, a_log_sigma)
    a0_3 = _chunk_edges(a_all[0], nch)
    a1_3 = _chunk_edges(a_all[1], nch)
    a2_3 = _chunk_edges(a_all[2], nch)

    aggP0 = _sc_spmm(g0, src3, dst3, a0_3, n=n, nch=nch)
    g1 = _tc_layer(aggP0, invi, invo, W0, b0, gamma0, beta0)
    aggP1 = _sc_spmm(g1, src3, dst3, a1_3, n=n, nch=nch)
    w2p = jnp.concatenate([W2, jnp.zeros_like(W2)], axis=1)
    p2 = _tc_layer(aggP1, invi, invo, W1, b1, gamma1, beta1, W2=w2p)
    aggP2 = _sc_spmm(p2, src3, dst3, a2_3, n=n, nch=nch)
    return _tc_final(aggP2, invi, b2)


# speedup vs baseline: 4.5690x; 1.3519x over previous
import functools

import jax
import jax.numpy as jnp
from jax import lax
from jax.experimental import pallas as pl
from jax.experimental.pallas import tpu as pltpu
from jax.experimental.pallas import tpu_sc as plsc

NC, NS, L = 2, 16, 16
NW = NC * NS


def _sc_mesh():
    return plsc.VectorSubcoreMesh(core_axis_name="c", subcore_axis_name="s")


def _zero_vmem(ref, rows, width):
    zero = jnp.zeros((L,), jnp.float32)

    def body(i, carry):
        for j in range(width // L):
            ref[i, pl.ds(j * L, L)] = zero
        return carry

    lax.fori_loop(0, rows, body, 0)


CB = 128


@functools.partial(jax.jit, static_argnames=("n", "nch"))
def _sc_spmm(g, src3, dst3, a3, *, n, nch):
    dw = g.shape[1]
    npad = ((n + NS * 128 - 1) // (NS * 128)) * (NS * 128)
    rpt = npad // NS
    wb = 128
    nwb = rpt // wb
    assert nch % 2 == 0

    @functools.partial(
        pl.kernel,
        out_type=jax.ShapeDtypeStruct((NC, npad, dw), jnp.float32),
        mesh=_sc_mesh(),
        compiler_params=pltpu.CompilerParams(needs_layout_passes=False),
        scratch_types=[
            pltpu.VMEM((nch, CB), jnp.int32),
            pltpu.VMEM((CB,), jnp.int32),
            pltpu.VMEM((CB,), jnp.int32),
            pltpu.VMEM((CB,), jnp.float32),
            pltpu.VMEM((CB,), jnp.float32),
            pltpu.VMEM((CB, dw), jnp.float32),
            pltpu.VMEM((CB, dw), jnp.float32),
            pltpu.VMEM_SHARED((npad, dw), jnp.float32),
            pltpu.SemaphoreType.DMA,
            pltpu.SemaphoreType.DMA,
            pltpu.SemaphoreType.DMA,
            pltpu.SemaphoreType.DMA,
        ],
    )
    def spmm_kernel(g_hbm, src_hbm, dst_hbm, a_hbm, out_hbm,
                    src_v, dst_b0, dst_b1, a_b0, a_b1, rows0, rows1,
                    acc, semg0, semg1, semi0, semi1):
        c = lax.axis_index("c")
        s = lax.axis_index("s")
        wid = c * NS + s
        row0 = s * rpt

        _zero_vmem(rows0, wb, dw)
        for i in range(nwb):
            pltpu.sync_copy(rows0, acc.at[pl.ds(row0 + i * wb, wb)])
        pltpu.sync_copy(src_hbm.at[wid], src_v)
        plsc.subcore_barrier()

        def prefetch(k, dst_b, a_b, rows_v, semg, semi):
            pltpu.async_copy(dst_hbm.at[wid, k], dst_b, semi)
            pltpu.async_copy(a_hbm.at[wid, k], a_b, semi)
            pltpu.async_copy(g_hbm.at[src_v.at[k]], rows_v, semg)

        def wait(k, dst_b, a_b, rows_v, semg, semi):
            pltpu.make_async_copy(dst_hbm.at[wid, k], dst_b, semi).wait()
            pltpu.make_async_copy(a_hbm.at[wid, k], a_b, semi).wait()
            pltpu.make_async_copy(g_hbm.at[src_v.at[k]], rows_v, semg).wait()

        def scale(rows_v, a_b):
            def edge(i, carry):
                ab = plsc.load_gather(a_b, [jnp.zeros((L,), jnp.int32) + i])
                for j in range(dw // L):
                    rows_v[i, pl.ds(j * L, L)] = (
                        rows_v[i, pl.ds(j * L, L)] * ab)
                return carry

            lax.fori_loop(0, CB, edge, 0)

        prefetch(0, dst_b0, a_b0, rows0, semg0, semi0)
        prefetch(1, dst_b1, a_b1, rows1, semg1, semi1)

        def pipe(j, carry):
            k0 = 2 * j
            wait(k0, dst_b0, a_b0, rows0, semg0, semi0)
            scale(rows0, a_b0)
            pltpu.sync_copy(rows0, acc.at[dst_b0], add=True)
            prefetch(jnp.minimum(k0 + 2, nch - 2),
                     dst_b0, a_b0, rows0, semg0, semi0)
            k1 = k0 + 1
            wait(k1, dst_b1, a_b1, rows1, semg1, semi1)
            scale(rows1, a_b1)
            pltpu.sync_copy(rows1, acc.at[dst_b1], add=True)
            prefetch(jnp.minimum(k1 + 2, nch - 1),
                     dst_b1, a_b1, rows1, semg1, semi1)
            return carry

        lax.fori_loop(0, nch // 2, pipe, 0)
        wait(nch - 2, dst_b0, a_b0, rows0, semg0, semi0)
        wait(nch - 1, dst_b1, a_b1, rows1, semg1, semi1)

        plsc.subcore_barrier()

        for i in range(nwb):
            r = row0 + i * wb
            pltpu.sync_copy(acc.at[pl.ds(r, wb)], rows0)
            pltpu.sync_copy(rows0, out_hbm.at[c, pl.ds(r, wb)])

    return spmm_kernel(g, src3, dst3, a3)


@functools.partial(jax.jit, static_argnames=("n", "nch"))
def _sc_degrees(src3, dst3, *, n, nch):
    dw = 128
    npad = ((n + NS * 128 - 1) // (NS * 128)) * (NS * 128)
    rpt = npad // NS
    wb = 128
    nwb = rpt // wb

    @functools.partial(
        pl.kernel,
        out_type=(
            jax.ShapeDtypeStruct((NC, npad, dw), jnp.float32),
            jax.ShapeDtypeStruct((NC, npad, dw), jnp.float32),
        ),
        mesh=_sc_mesh(),
        compiler_params=pltpu.CompilerParams(needs_layout_passes=False),
        scratch_types=[
            pltpu.VMEM((CB,), jnp.int32),
            pltpu.VMEM((CB,), jnp.int32),
            pltpu.VMEM((CB, dw), jnp.float32),
            pltpu.VMEM((wb, dw), jnp.float32),
            pltpu.VMEM_SHARED((npad, dw), jnp.float32),
            pltpu.SemaphoreType.DMA,
            pltpu.SemaphoreType.DMA,
        ],
    )
    def deg_kernel(src_hbm, dst_hbm, outd_hbm, outi_hbm,
                   idx_b0, idx_b1, ones_v, wb_v, acc, sem0, sem1):
        c = lax.axis_index("c")
        s = lax.axis_index("s")
        wid = c * NS + s
        row0 = s * rpt

        one = jnp.ones((L,), jnp.float32)

        def fill(i, carry):
            for j in range(dw // L):
                ones_v[i, pl.ds(j * L, L)] = one
            return carry

        lax.fori_loop(0, CB, fill, 0)

        def one_direction(idx_hbm, out_hbm):
            _zero_vmem(wb_v, wb, dw)
            for i in range(nwb):
                pltpu.sync_copy(wb_v, acc.at[pl.ds(row0 + i * wb, wb)])
            plsc.subcore_barrier()

            pltpu.async_copy(idx_hbm.at[wid, 0], idx_b0, sem0)
            pltpu.async_copy(idx_hbm.at[wid, 1], idx_b1, sem1)

            def pipe(j, carry):
                k0 = 2 * j
                pltpu.make_async_copy(
                    idx_hbm.at[wid, k0], idx_b0, sem0).wait()
                pltpu.sync_copy(ones_v, acc.at[idx_b0], add=True)
                pltpu.async_copy(
                    idx_hbm.at[wid, jnp.minimum(k0 + 2, nch - 2)],
                    idx_b0, sem0)
                k1 = k0 + 1
                pltpu.make_async_copy(
                    idx_hbm.at[wid, k1], idx_b1, sem1).wait()
                pltpu.sync_copy(ones_v, acc.at[idx_b1], add=True)
                pltpu.async_copy(
                    idx_hbm.at[wid, jnp.minimum(k1 + 2, nch - 1)],
                    idx_b1, sem1)
                return carry

            lax.fori_loop(0, nch // 2, pipe, 0)
            pltpu.make_async_copy(idx_hbm.at[wid, nch - 2], idx_b0, sem0).wait()
            pltpu.make_async_copy(idx_hbm.at[wid, nch - 1], idx_b1, sem1).wait()
            plsc.subcore_barrier()

            for i in range(nwb):
                r = row0 + i * wb
                pltpu.sync_copy(acc.at[pl.ds(r, wb)], wb_v)
                pltpu.sync_copy(wb_v, out_hbm.at[c, pl.ds(r, wb)])
            plsc.subcore_barrier()

        one_direction(src_hbm, outd_hbm)
        one_direction(dst_hbm, outi_hbm)

    return deg_kernel(src3, dst3)


def _chunk_edges(v, nch, pad_val=0):
    pad = NW * nch * CB - v.shape[0]
    return jnp.concatenate(
        [v, jnp.full((pad,), pad_val, v.dtype)]).reshape(NW, nch, CB)


def _tc_prep(degoP, degiP, x, eps, a_mu, a_log_sigma):
    n, d = x.shape

    def body(dego_ref, degi_ref, x_ref, eps_ref, amu_ref, als_ref,
             invo_ref, invi_ref, g0_ref, a_ref):
        dego = dego_ref[0, :n, 0:1] + dego_ref[1, :n, 0:1]
        degi = degi_ref[0, :n, 0:1] + degi_ref[1, :n, 0:1]
        invo = lax.rsqrt(jnp.where(dego > 0.0, dego, 1.0))
        invi = lax.rsqrt(jnp.where(degi > 0.0, degi, 1.0))
        invo_ref[...] = invo
        invi_ref[...] = invi
        g0_ref[...] = x_ref[...] * invo
        sigma = jnp.exp(als_ref[0, 0])
        a_ref[...] = amu_ref[0, 0] + sigma * eps_ref[...]

    return pl.pallas_call(
        body,
        out_shape=(
            jax.ShapeDtypeStruct((n, 1), jnp.float32),
            jax.ShapeDtypeStruct((n, 1), jnp.float32),
            jax.ShapeDtypeStruct((n, d), jnp.float32),
            jax.ShapeDtypeStruct(eps.shape, jnp.float32),
        ),
    )(degoP, degiP, x, eps, a_mu.reshape(1, 1), a_log_sigma.reshape(1, 1))


def _tc_layer(aggP, invi, invo, W, b, gamma, beta, W2=None):
    n = invi.shape[0]
    hdim = W.shape[1]
    odim = hdim if W2 is None else W2.shape[1]

    def body(*refs):
        if W2 is None:
            aggp_ref, invi_ref, invo_ref, w_ref, b_ref, g_ref, be_ref, o_ref = refs
        else:
            (aggp_ref, invi_ref, invo_ref, w_ref, b_ref, g_ref, be_ref,
             w2_ref, o_ref) = refs
        agg = (aggp_ref[0, :n] + aggp_ref[1, :n]) * invi_ref[...]
        tmat = jnp.dot(agg, w_ref[...], preferred_element_type=jnp.float32)
        tmat = tmat + b_ref[...]
        tmat = jnp.maximum(tmat, 0.0)
        mu = jnp.mean(tmat, axis=0, keepdims=True)
        var = jnp.mean((tmat - mu) * (tmat - mu), axis=0, keepdims=True)
        h = (tmat - mu) * lax.rsqrt(var + 1e-5) * g_ref[...] + be_ref[...]
        h = h * invo_ref[...]
        if W2 is None:
            o_ref[...] = h
        else:
            o_ref[...] = jnp.dot(h, w2_ref[...],
                                 preferred_element_type=jnp.float32)

    args = [aggP, invi, invo, W, b.reshape(1, hdim), gamma.reshape(1, hdim),
            beta.reshape(1, hdim)]
    if W2 is not None:
        args.append(W2)
    return pl.pallas_call(
        body,
        out_shape=jax.ShapeDtypeStruct((n, odim), jnp.float32),
    )(*args)


def _tc_final(aggP, invi, b2):
    n = invi.shape[0]
    cdim = b2.shape[0]

    def body(aggp_ref, invi_ref, b_ref, o_ref):
        z = ((aggp_ref[0, :n, :cdim] + aggp_ref[1, :n, :cdim])
             * invi_ref[...] + b_ref[...])
        m = jnp.max(z, axis=1, keepdims=True)
        ez = jnp.exp(z - m)
        o_ref[...] = ez / jnp.sum(ez, axis=1, keepdims=True)

    return pl.pallas_call(
        body,
        out_shape=jax.ShapeDtypeStruct((n, cdim), jnp.float32),
    )(aggP, invi, b2.reshape(1, cdim))


def kernel(x, edge_index, W0, b0, W1, b1, W2, b2,
           gamma0, beta0, gamma1, beta1, a_mu, a_log_sigma):
    n, d = x.shape
    e = edge_index.shape[1]
    src = edge_index[0].astype(jnp.int32)
    dst = edge_index[1].astype(jnp.int32)

    eps = jnp.stack([
        jax.random.normal(jax.random.key(101), (e,), dtype=jnp.float32),
        jax.random.normal(jax.random.key(102), (e,), dtype=jnp.float32),
        jax.random.normal(jax.random.key(103), (e,), dtype=jnp.float32),
    ])

    nch = -(-e // (NW * CB * 2)) * 2
    src3 = _chunk_edges(src, nch)
    src3c = _chunk_edges(src, nch, pad_val=n)
    dst3 = _chunk_edges(dst, nch, pad_val=n)
    degoP, degiP = _sc_degrees(src3c, dst3, n=n, nch=nch)
    invo, invi, g0, a_all = _tc_prep(degoP, degiP, x, eps, a_mu, a_log_sigma)
    a0_3 = _chunk_edges(a_all[0], nch)
    a1_3 = _chunk_edges(a_all[1], nch)
    a2_3 = _chunk_edges(a_all[2], nch)

    aggP0 = _sc_spmm(g0, src3, dst3, a0_3, n=n, nch=nch)
    g1 = _tc_layer(aggP0, invi, invo, W0, b0, gamma0, beta0)
    aggP1 = _sc_spmm(g1, src3, dst3, a1_3, n=n, nch=nch)
    w2p = jnp.concatenate([W2, jnp.zeros_like(W2)], axis=1)
    p2 = _tc_layer(aggP1, invi, invo, W1, b1, gamma1, beta1, W2=w2p)
    aggP2 = _sc_spmm(p2, src3, dst3, a2_3, n=n, nch=nch)
    return _tc_final(aggP2, invi, b2)
